# baseline (speedup 1.0000x reference)
import functools

import jax
import jax.numpy as jnp
from jax import lax
from jax.experimental import pallas as pl
from jax.experimental.pallas import tpu as pltpu
from jax.experimental.pallas import tpu_sc as plsc

D_MODEL = 64
_DP = 128

_info = plsc.get_sparse_core_info()
_NC, _NS = _info.num_cores, _info.num_subcores
_NW = _NC * _NS

_CHUNK = 128
_NBUF = 2


def _make_emb(n_rows: int, d: int):
    rows_per_w = n_rows // _NW
    n_chunks = rows_per_w // _CHUNK
    assert n_chunks % _NBUF == 0
    mesh = plsc.VectorSubcoreMesh(core_axis_name="c", subcore_axis_name="s")

    @functools.partial(
        pl.kernel,
        mesh=mesh,
        out_type=jax.ShapeDtypeStruct((n_rows, d), jnp.float32),
        compiler_params=pltpu.CompilerParams(needs_layout_passes=False),
        scratch_types=[
            pltpu.VMEM((n_chunks, _CHUNK), jnp.int32),
            pltpu.VMEM((_NBUF, _CHUNK, _DP), jnp.float32),
            pltpu.VMEM((_NBUF, _CHUNK, D_MODEL), jnp.float32),
        ]
        + [pltpu.SemaphoreType.DMA] * (2 * _NBUF),
    )
    def emb(idx_hbm, table_hbm, out_hbm, idx_v, rows_v, rows64_v, *sems):
        gsem = sems[:_NBUF]
        wsem = sems[_NBUF:]
        wid = lax.axis_index("s") * _NC + lax.axis_index("c")
        chunk_base = wid * n_chunks
        row_base = chunk_base * _CHUNK

        pltpu.sync_copy(idx_hbm.at[pl.ds(chunk_base, n_chunks)], idx_v)

        def fire_chunk(c, b):
            pltpu.async_copy(table_hbm.at[idx_v.at[c]], rows_v.at[b], gsem[b])

        for b in range(_NBUF):
            fire_chunk(b, b)

        def group(g, carry):
            for b in range(_NBUF):
                c = g * _NBUF + b
                row_off = row_base + c * _CHUNK
                out_slice = out_hbm.at[pl.ds(row_off, _CHUNK)]
                pltpu.make_async_copy(
                    table_hbm.at[idx_v.at[c]], rows_v.at[b], gsem[b]
                ).wait()
                @plsc.parallel_loop(0, _CHUNK, unroll=8)
                def _compact(i):
                    for kk in range(d // 16):
                        rows64_v[b, i, pl.ds(kk * 16, 16)] = rows_v[
                            b, i, pl.ds(kk * 16, 16)
                        ]

                pltpu.async_copy(rows64_v.at[b], out_slice, wsem[b])
                nc = c + _NBUF

                @pl.when(nc < n_chunks)
                def _():
                    fire_chunk(nc, b)

                @pl.when(jnp.logical_and(c + 1 >= _NBUF, c + 1 < n_chunks))
                def _():
                    pc = c + 1 - _NBUF
                    pltpu.make_async_copy(
                        rows64_v.at[(b + 1) % _NBUF],
                        out_hbm.at[pl.ds(row_base + pc * _CHUNK, _CHUNK)],
                        wsem[(b + 1) % _NBUF],
                    ).wait()

            return carry

        lax.fori_loop(0, n_chunks // _NBUF, group, 0)

        for b in range(_NBUF):
            c = n_chunks - _NBUF + b
            pltpu.make_async_copy(
                rows64_v.at[b],
                out_hbm.at[pl.ds(row_base + c * _CHUNK, _CHUNK)],
                wsem[b],
            ).wait()

    return emb


def kernel(x, table):
    b, s = x.shape
    n = b * s
    idx2d = x.reshape(n // _CHUNK, _CHUNK).astype(jnp.int32)
    table_p = jnp.pad(table, ((0, 0), (0, _DP - D_MODEL)))
    out = _make_emb(n, D_MODEL)(idx2d, table_p)
    return out.reshape(b, s, D_MODEL)


# speedup vs baseline: 1.3034x; 1.3034x over previous
import functools

import jax
import jax.numpy as jnp
from jax import lax
from jax.experimental import pallas as pl
from jax.experimental.pallas import tpu as pltpu
from jax.experimental.pallas import tpu_sc as plsc

D_MODEL = 64

_info = plsc.get_sparse_core_info()
_NC, _NS = _info.num_cores, _info.num_subcores
_NW = _NC * _NS

_CHUNK = 128
_NBUF = 4


def _make_emb(n_rows: int, d: int):
    rows_per_w = n_rows // _NW
    n_chunks = rows_per_w // _CHUNK
    assert n_chunks % _NBUF == 0
    mesh = plsc.VectorSubcoreMesh(core_axis_name="c", subcore_axis_name="s")

    @functools.partial(
        pl.kernel,
        mesh=mesh,
        out_type=jax.ShapeDtypeStruct((n_rows, d), jnp.float32),
        compiler_params=pltpu.CompilerParams(needs_layout_passes=False),
        scratch_types=[
            pltpu.VMEM((n_chunks, _CHUNK), jnp.int32),
            pltpu.VMEM((_NBUF, _CHUNK, d), jnp.float32),
        ]
        + [pltpu.SemaphoreType.DMA] * (2 * _NBUF),
    )
    def emb(idx_hbm, table_hbm, out_hbm, idx_v, rows_v, *sems):
        gsem = sems[:_NBUF]
        wsem = sems[_NBUF:]
        wid = lax.axis_index("s") * _NC + lax.axis_index("c")
        chunk_base = wid * n_chunks
        row_base = chunk_base * _CHUNK

        pltpu.sync_copy(idx_hbm.at[pl.ds(chunk_base, n_chunks)], idx_v)

        def fire_chunk(c, b):
            def row16(v, carry):
                vec = idx_v[c, pl.ds(v * 16, 16)]
                for j in range(16):
                    t = vec[j]
                    pltpu.async_copy(
                        table_hbm.at[t // 8, t % 8],
                        rows_v.at[b, v * 16 + j],
                        gsem[b],
                    )
                return carry

            lax.fori_loop(0, _CHUNK // 16, row16, 0)

        for b in range(_NBUF):
            fire_chunk(b, b)

        def group(g, carry):
            for b in range(_NBUF):
                c = g * _NBUF + b
                row_off = row_base + c * _CHUNK
                out_slice = out_hbm.at[pl.ds(row_off, _CHUNK)]
                pltpu.make_async_copy(out_slice, rows_v.at[b], gsem[b]).wait()
                pltpu.async_copy(rows_v.at[b], out_slice, wsem[b])
                nc = c + _NBUF

                @pl.when(nc < n_chunks)
                def _():
                    pltpu.make_async_copy(
                        rows_v.at[b], out_slice, wsem[b]
                    ).wait()
                    fire_chunk(nc, b)

            return carry

        lax.fori_loop(0, n_chunks // _NBUF, group, 0)

        for b in range(_NBUF):
            c = n_chunks - _NBUF + b
            pltpu.make_async_copy(
                rows_v.at[b],
                out_hbm.at[pl.ds(row_base + c * _CHUNK, _CHUNK)],
                wsem[b],
            ).wait()

    return emb


def kernel(x, table):
    b, s = x.shape
    n = b * s
    v = table.shape[0]
    idx2d = x.reshape(n // _CHUNK, _CHUNK).astype(jnp.int32)
    table3 = table.reshape(v // 8, 8, D_MODEL)
    out = _make_emb(n, D_MODEL)(idx2d, table3)
    return out.reshape(b, s, D_MODEL)


# speedup vs baseline: 1.4458x; 1.1093x over previous
import functools

import jax
import jax.numpy as jnp
from jax import lax
from jax.experimental import pallas as pl
from jax.experimental.pallas import tpu as pltpu
from jax.experimental.pallas import tpu_sc as plsc

D_MODEL = 64

_info = plsc.get_sparse_core_info()
_NC, _NS = _info.num_cores, _info.num_subcores
_NW = _NC * _NS

_CHUNK = 128
_NBUF = 4


def _make_emb(n_rows: int, d: int):
    rows_per_w = n_rows // _NW
    n_chunks = rows_per_w // _CHUNK
    assert n_chunks % _NBUF == 0
    mesh = plsc.VectorSubcoreMesh(core_axis_name="c", subcore_axis_name="s")

    @functools.partial(
        pl.kernel,
        mesh=mesh,
        out_type=jax.ShapeDtypeStruct((n_rows, d), jnp.float32),
        compiler_params=pltpu.CompilerParams(needs_layout_passes=False),
        scratch_types=[
            pltpu.VMEM((n_chunks, _CHUNK), jnp.int32),
            pltpu.VMEM((_NBUF, _CHUNK, d), jnp.float32),
        ]
        + [pltpu.SemaphoreType.DMA] * (2 * _NBUF),
    )
    def emb(idx_hbm, table_hbm, out_hbm, idx_v, rows_v, *sems):
        gsem = sems[:_NBUF]
        wsem = sems[_NBUF:]
        wid = lax.axis_index("s") * _NC + lax.axis_index("c")
        chunk_base = wid * n_chunks
        row_base = chunk_base * _CHUNK

        pltpu.sync_copy(idx_hbm.at[pl.ds(chunk_base, n_chunks)], idx_v)

        def fire_chunk(c, b):
            def row16(v, carry):
                vec = idx_v[c, pl.ds(v * 16, 16)]
                vq = lax.shift_right_logical(vec, 3)
                vr = lax.bitwise_and(vec, 7)
                for j in range(16):
                    pltpu.async_copy(
                        table_hbm.at[vq[j], vr[j]],
                        rows_v.at[b, v * 16 + j],
                        gsem[b],
                    )
                return carry

            lax.fori_loop(0, _CHUNK // 16, row16, 0)

        for b in range(_NBUF):
            fire_chunk(b, b)

        def group(g, carry):
            for b in range(_NBUF):
                c = g * _NBUF + b
                row_off = row_base + c * _CHUNK
                out_slice = out_hbm.at[pl.ds(row_off, _CHUNK)]
                pltpu.make_async_copy(out_slice, rows_v.at[b], gsem[b]).wait()
                pltpu.async_copy(rows_v.at[b], out_slice, wsem[b])
                nc = c + _NBUF

                @pl.when(nc < n_chunks)
                def _():
                    pltpu.make_async_copy(
                        rows_v.at[b], out_slice, wsem[b]
                    ).wait()
                    fire_chunk(nc, b)

            return carry

        lax.fori_loop(0, n_chunks // _NBUF, group, 0)

        for b in range(_NBUF):
            c = n_chunks - _NBUF + b
            pltpu.make_async_copy(
                rows_v.at[b],
                out_hbm.at[pl.ds(row_base + c * _CHUNK, _CHUNK)],
                wsem[b],
            ).wait()

    return emb


def kernel(x, table):
    b, s = x.shape
    n = b * s
    v = table.shape[0]
    idx2d = x.reshape(n // _CHUNK, _CHUNK).astype(jnp.int32)
    table3 = table.reshape(v // 8, 8, D_MODEL)
    out = _make_emb(n, D_MODEL)(idx2d, table3)
    return out.reshape(b, s, D_MODEL)


# speedup vs baseline: 1.4459x; 1.0001x over previous
import functools

import jax
import jax.numpy as jnp
from jax import lax
from jax.experimental import pallas as pl
from jax.experimental.pallas import tpu as pltpu
from jax.experimental.pallas import tpu_sc as plsc

D_MODEL = 64

_info = plsc.get_sparse_core_info()
_NC, _NS = _info.num_cores, _info.num_subcores
_NW = _NC * _NS

_CHUNK = 128
_NBUF = 4


def _make_emb(n_rows: int, d: int):
    rows_per_w = n_rows // _NW
    n_chunks = rows_per_w // _CHUNK
    assert n_chunks % _NBUF == 0
    mesh = plsc.VectorSubcoreMesh(core_axis_name="c", subcore_axis_name="s")

    @functools.partial(
        pl.kernel,
        mesh=mesh,
        out_type=jax.ShapeDtypeStruct((n_rows, d), jnp.float32),
        compiler_params=pltpu.CompilerParams(needs_layout_passes=False),
        scratch_types=[
            pltpu.VMEM((n_chunks, _CHUNK), jnp.int32),
            pltpu.VMEM((_NBUF, _CHUNK, d), jnp.float32),
        ]
        + [pltpu.SemaphoreType.DMA] * (2 * _NBUF),
    )
    def emb(idx_hbm, table_hbm, out_hbm, idx_v, rows_v, *sems):
        gsem = sems[:_NBUF]
        wsem = sems[_NBUF:]
        wid = lax.axis_index("s") * _NC + lax.axis_index("c")
        chunk_base = wid * n_chunks
        row_base = chunk_base * _CHUNK

        pltpu.sync_copy(idx_hbm.at[pl.ds(chunk_base, n_chunks)], idx_v)

        def fire_chunk(c, b):
            def row16(v, carry):
                vec = idx_v[c, pl.ds(v * 16, 16)]
                for j in range(16):
                    t = vec[j]
                    pltpu.async_copy(
                        table_hbm.at[
                            lax.shift_right_logical(t, 3),
                            lax.bitwise_and(t, 7),
                        ],
                        rows_v.at[b, v * 16 + j],
                        gsem[b],
                    )
                return carry

            lax.fori_loop(0, _CHUNK // 16, row16, 0)

        for b in range(_NBUF):
            fire_chunk(b, b)

        def group(g, carry):
            for b in range(_NBUF):
                c = g * _NBUF + b
                row_off = row_base + c * _CHUNK
                out_slice = out_hbm.at[pl.ds(row_off, _CHUNK)]
                pltpu.make_async_copy(out_slice, rows_v.at[b], gsem[b]).wait()
                pltpu.async_copy(rows_v.at[b], out_slice, wsem[b])
                nc = c + _NBUF

                @pl.when(nc < n_chunks)
                def _():
                    pltpu.make_async_copy(
                        rows_v.at[b], out_slice, wsem[b]
                    ).wait()
                    fire_chunk(nc, b)

            return carry

        lax.fori_loop(0, n_chunks // _NBUF, group, 0)

        for b in range(_NBUF):
            c = n_chunks - _NBUF + b
            pltpu.make_async_copy(
                rows_v.at[b],
                out_hbm.at[pl.ds(row_base + c * _CHUNK, _CHUNK)],
                wsem[b],
            ).wait()

    return emb


def kernel(x, table):
    b, s = x.shape
    n = b * s
    v = table.shape[0]
    idx2d = x.reshape(n // _CHUNK, _CHUNK).astype(jnp.int32)
    table3 = table.reshape(v // 8, 8, D_MODEL)
    out = _make_emb(n, D_MODEL)(idx2d, table3)
    return out.reshape(b, s, D_MODEL)
